# Initial kernel scaffold; baseline (speedup 1.0000x reference)
import jax
import jax.numpy as jnp
from jax.experimental import pallas as pl


def kernel(x, edge_index, batch, W1, b1, W2, b2):
    raise NotImplementedError("write your pallas kernel here")


# speedup vs baseline: 30.7875x; 30.7875x over previous
import functools

import jax
import jax.numpy as jnp
from jax import lax
from jax.experimental import pallas as pl
from jax.experimental.pallas import tpu as pltpu
from jax.experimental.pallas import tpu_sc as plsc

N = 10000
NP = 10240
E = 320000
F = 128
NG = 64
SUB = 128
KSUB = 10
INC = SUB * KSUB
NIN = E // INC
PFLAT = NG * NP
TECN = NP // 16


def _rsqrt_newton(d):
    i = lax.bitcast_convert_type(d, jnp.int32)
    i = 0x5F3759DF - lax.shift_right_logical(i, 1)
    y = lax.bitcast_convert_type(i, jnp.float32)
    for _ in range(3):
        y = y * (1.5 - 0.5 * d * y * y)
    return y


def _relayout_1d_to_2d(buf1, buf2):
    def outer(k, _):
        def inner(g, _):
            buf2[k, pl.ds(g * 16, 16)] = buf1[pl.ds(k * SUB + g * 16, 16)]
            return ()
        return lax.fori_loop(0, SUB // 16, inner, ())
    lax.fori_loop(0, KSUB, outer, ())


def _mk_edge_stats():
    mesh = plsc.VectorSubcoreMesh(core_axis_name="c", subcore_axis_name="s")

    @functools.partial(
        pl.kernel,
        mesh=mesh,
        compiler_params=pltpu.CompilerParams(needs_layout_passes=False),
        out_type=[
            jax.ShapeDtypeStruct((2 * NP,), jnp.float32),
            jax.ShapeDtypeStruct((2 * PFLAT,), jnp.float32),
        ],
        scratch_types=[
            pltpu.VMEM((INC,), jnp.int32),
            pltpu.VMEM((INC,), jnp.int32),
            pltpu.VMEM((KSUB, SUB), jnp.int32),
            pltpu.VMEM((KSUB, SUB), jnp.float32),
            pltpu.VMEM((KSUB, SUB), jnp.int32),
            pltpu.VMEM((KSUB, SUB), jnp.float32),
            pltpu.VMEM((NP,), jnp.int32),
            pltpu.VMEM((NP,), jnp.float32),
            pltpu.VMEM((2560,), jnp.float32),
            pltpu.VMEM_SHARED((NP,), jnp.float32),
            pltpu.VMEM_SHARED((PFLAT,), jnp.float32),
        ],
    )
    def kern(src_h, dst_h, batch, dinv_out, p_out,
             src_sf, dst_sf, dst_full, val_full, flat_full, ones_full,
             batch_tbl, dinv_tbl, zbuf, deg_sh, p_sh):
        cid = lax.axis_index("c")
        sid = lax.axis_index("s")
        wid = cid * 16 + sid
        z16 = jnp.zeros((16,), jnp.float32)
        o16 = jnp.ones((16,), jnp.float32)

        def zb(i, _):
            zbuf[pl.ds(i * 16, 16)] = z16
            return ()
        lax.fori_loop(0, 160, zb, ())

        def ob(k, _):
            def obi(g, _):
                ones_full[k, pl.ds(g * 16, 16)] = o16
                return ()
            return lax.fori_loop(0, 8, obi, ())
        lax.fori_loop(0, KSUB, ob, ())

        pltpu.sync_copy(batch, batch_tbl)
        pltpu.sync_copy(zbuf.at[pl.ds(0, TECN)],
                        deg_sh.at[pl.ds(sid * TECN, TECN)])
        for q in range(16):
            pltpu.sync_copy(zbuf, p_sh.at[pl.ds(sid * 16 * 2560 + q * 2560, 2560)])
        plsc.subcore_barrier()

        def deg_body(j, _):
            c = sid + 16 * j

            @pl.when(c < NIN)
            def _():
                pltpu.sync_copy(dst_h.at[pl.ds(c * INC, INC)], dst_sf)
                _relayout_1d_to_2d(dst_sf, dst_full)

                def stream_k(k, _):
                    pltpu.sync_copy(ones_full.at[k],
                                    deg_sh.at[dst_full.at[k]], add=True)
                    return ()
                lax.fori_loop(0, KSUB, stream_k, ())
            return ()
        lax.fori_loop(0, 16, deg_body, ())
        plsc.subcore_barrier()

        pltpu.sync_copy(deg_sh, dinv_tbl)

        def dinv_body(i, _):
            d = dinv_tbl[pl.ds(i * 16, 16)] + 1.0
            dinv_tbl[pl.ds(i * 16, 16)] = _rsqrt_newton(d)
            return ()
        lax.fori_loop(0, NP // 16, dinv_body, ())

        def d1_body(i, _):
            zbuf[pl.ds(i * 16, 16)] = dinv_tbl[pl.ds(sid * TECN + i * 16, 16)]
            return ()
        lax.fori_loop(0, TECN // 16, d1_body, ())
        pltpu.sync_copy(zbuf.at[pl.ds(0, TECN)],
                        dinv_out.at[pl.ds(cid * NP + sid * TECN, TECN)])

        def norm_body(j, _):
            c = wid + 32 * j

            @pl.when(c < NIN)
            def _():
                pltpu.sync_copy(src_h.at[pl.ds(c * INC, INC)], src_sf)
                pltpu.sync_copy(dst_h.at[pl.ds(c * INC, INC)], dst_sf)

                def per_k(k, _):
                    def per_g(g, _):
                        sl16 = pl.ds(k * SUB + g * 16, 16)
                        sv = src_sf[sl16]
                        dv = dst_sf[sl16]
                        a = plsc.load_gather(dinv_tbl, [sv])
                        b = plsc.load_gather(dinv_tbl, [dv])
                        bb = plsc.load_gather(batch_tbl, [dv])
                        sl = pl.ds(g * 16, 16)
                        val_full[k, sl] = a * b
                        flat_full[k, sl] = bb * NP + sv
                        return ()
                    lax.fori_loop(0, 8, per_g, ())
                    pltpu.sync_copy(val_full.at[k],
                                    p_sh.at[flat_full.at[k]], add=True)
                    return ()
                lax.fori_loop(0, KSUB, per_k, ())
            return ()
        lax.fori_loop(0, 8, norm_body, ())
        plsc.subcore_barrier()

        for q in range(16):
            off = sid * 16 * 2560 + q * 2560
            pltpu.sync_copy(p_sh.at[pl.ds(off, 2560)],
                            p_out.at[pl.ds(cid * PFLAT + off, 2560)])

    return kern


def _mk_aggregate():
    mesh = plsc.VectorSubcoreMesh(core_axis_name="c", subcore_axis_name="s")

    @functools.partial(
        pl.kernel,
        mesh=mesh,
        compiler_params=pltpu.CompilerParams(needs_layout_passes=False),
        out_type=jax.ShapeDtypeStruct((2 * NP, F), jnp.float32),
        scratch_types=[
            pltpu.VMEM((INC,), jnp.int32),
            pltpu.VMEM((INC,), jnp.int32),
            pltpu.VMEM((KSUB, SUB), jnp.int32),
            pltpu.VMEM((SUB, F), jnp.float32),
            pltpu.VMEM_SHARED((NP, F), jnp.float32),
        ],
    )
    def kern(src_h, dst_h, gs_in, e_out,
             src_sf, dst_sf, dst_full, rows, acc_sh):
        cid = lax.axis_index("c")
        sid = lax.axis_index("s")
        wid = cid * 16 + sid
        z16 = jnp.zeros((16,), jnp.float32)

        def zr(r, _):
            def zri(t, _):
                rows[r, pl.ds(t * 16, 16)] = z16
                return ()
            return lax.fori_loop(0, 8, zri, ())
        lax.fori_loop(0, SUB, zr, ())
        for q in range(TECN // SUB):
            pltpu.sync_copy(rows, acc_sh.at[pl.ds(sid * TECN + q * SUB, SUB), :])
        plsc.subcore_barrier()

        def body(j, _):
            c = wid + 32 * j

            @pl.when(c < NIN)
            def _():
                pltpu.sync_copy(src_h.at[pl.ds(c * INC, INC)], src_sf)
                pltpu.sync_copy(dst_h.at[pl.ds(c * INC, INC)], dst_sf)
                _relayout_1d_to_2d(dst_sf, dst_full)

                def per_k(k, _):
                    pltpu.sync_copy(gs_in.at[src_sf.at[pl.ds(k * SUB, SUB)]],
                                    rows)
                    pltpu.sync_copy(rows, acc_sh.at[dst_full.at[k]], add=True)
                    return ()
                lax.fori_loop(0, KSUB, per_k, ())
            return ()
        lax.fori_loop(0, 8, body, ())
        plsc.subcore_barrier()

        pltpu.sync_copy(acc_sh.at[pl.ds(sid * TECN, TECN), :],
                        e_out.at[pl.ds(cid * NP + sid * TECN, TECN), :])

    return kern


def _xw_body(x_ref, w_ref, d1_ref, o_ref):
    o_ref[...] = d1_ref[...] * jnp.dot(x_ref[...], w_ref[...],
                                       preferred_element_type=jnp.float32)


def _mk_xw():
    return pl.pallas_call(
        _xw_body,
        grid=(NP // 1024,),
        in_specs=[
            pl.BlockSpec((1024, F), lambda i: (i, 0)),
            pl.BlockSpec((F, F), lambda i: (0, 0)),
            pl.BlockSpec((1024, 1), lambda i: (i, 0)),
        ],
        out_specs=pl.BlockSpec((1024, F), lambda i: (i, 0)),
        out_shape=jax.ShapeDtypeStruct((NP, F), jnp.float32),
    )


def _final_body(gs_ref, e_ref, d1_ref, bt_ref, p_ref, b1_ref, w2_ref, b2_ref,
                o_ref, accp, accs, accc):
    i = pl.program_id(0)

    @pl.when(i == 0)
    def _():
        accp[...] = jnp.zeros_like(accp)
        accs[...] = jnp.zeros_like(accs)
        accc[...] = jnp.zeros_like(accc)

    gsb = gs_ref[...]
    eb = e_ref[0] + e_ref[1]
    d1 = d1_ref[...]
    hb = jax.nn.relu(d1 * (gsb + eb) + b1_ref[...])
    pb = p_ref[0] + p_ref[1]
    msel = (lax.broadcasted_iota(jnp.int32, (1024, NG), 1)
            == bt_ref[...]).astype(jnp.float32)
    dn = (((0,), (0,)), ((), ()))
    accp[...] += jnp.dot(pb, hb, preferred_element_type=jnp.float32)
    accs[...] += lax.dot_general(msel, (d1 * d1) * hb, dn,
                                 preferred_element_type=jnp.float32)
    accc[...] += lax.dot_general(msel, jnp.ones((1024, F), jnp.float32), dn,
                                 preferred_element_type=jnp.float32)

    @pl.when(i == NP // 1024 - 1)
    def _():
        pooled = (accp[...] + accs[...]) / jnp.maximum(accc[...], 1.0)
        logits = jnp.dot(pooled, w2_ref[...],
                         preferred_element_type=jnp.float32) + b2_ref[...]
        m = jnp.max(logits, axis=1, keepdims=True)
        s = logits - m
        o_ref[...] = s - jnp.log(jnp.sum(jnp.exp(s), axis=1, keepdims=True))


def _mk_final():
    nb = NP // 1024
    return pl.pallas_call(
        _final_body,
        grid=(nb,),
        in_specs=[
            pl.BlockSpec((1024, F), lambda i: (i, 0)),
            pl.BlockSpec((2, 1024, F), lambda i: (0, i, 0)),
            pl.BlockSpec((1024, 1), lambda i: (i, 0)),
            pl.BlockSpec((1024, 1), lambda i: (i, 0)),
            pl.BlockSpec((2, NG, 1024), lambda i: (0, 0, i)),
            pl.BlockSpec((1, F), lambda i: (0, 0)),
            pl.BlockSpec((F, F), lambda i: (0, 0)),
            pl.BlockSpec((1, F), lambda i: (0, 0)),
        ],
        out_specs=pl.BlockSpec((NG, F), lambda i: (0, 0)),
        out_shape=jax.ShapeDtypeStruct((NG, F), jnp.float32),
        scratch_shapes=[
            pltpu.VMEM((NG, F), jnp.float32),
            pltpu.VMEM((NG, F), jnp.float32),
            pltpu.VMEM((NG, F), jnp.float32),
        ],
    )


_edge_stats = _mk_edge_stats()
_aggregate = _mk_aggregate()
_xw = _mk_xw()
_final = _mk_final()


def kernel(x, edge_index, batch, W1, b1, W2, b2):
    src = edge_index[0]
    dst = edge_index[1]
    x_pad = jnp.pad(x, ((0, NP - N), (0, 0)))
    batch_pad = jnp.pad(batch, (0, NP - N), constant_values=NG)

    dinv, p_part = _edge_stats(src, dst, batch_pad)
    d1col = dinv[:NP][:, None]
    gs = _xw(x_pad, W1, d1col)
    e_part = _aggregate(src, dst, gs)

    return _final(gs, e_part.reshape(2, NP, F), d1col, batch_pad[:, None],
                  p_part.reshape(2, NG, NP),
                  b1.reshape(1, F), W2, b2.reshape(1, F))


# speedup vs baseline: 36.2290x; 1.1767x over previous
import functools

import jax
import jax.numpy as jnp
from jax import lax
from jax.experimental import pallas as pl
from jax.experimental.pallas import tpu as pltpu
from jax.experimental.pallas import tpu_sc as plsc

N = 10000
NP = 10240
E = 320000
F = 128
NG = 64
SUB = 128
KSUB = 10
INC = SUB * KSUB
NIN = E // INC
PFLAT = NG * NP
TECN = NP // 16


def _rsqrt_newton(d):
    i = lax.bitcast_convert_type(d, jnp.int32)
    i = 0x5F3759DF - lax.shift_right_logical(i, 1)
    y = lax.bitcast_convert_type(i, jnp.float32)
    for _ in range(3):
        y = y * (1.5 - 0.5 * d * y * y)
    return y


def _relayout_1d_to_2d(buf1, buf2):
    def outer(k, _):
        def inner(g, _):
            buf2[k, pl.ds(g * 16, 16)] = buf1[pl.ds(k * SUB + g * 16, 16)]
            return ()
        return lax.fori_loop(0, SUB // 16, inner, ())
    lax.fori_loop(0, KSUB, outer, ())


def _mk_edge_stats():
    mesh = plsc.VectorSubcoreMesh(core_axis_name="c", subcore_axis_name="s")

    @functools.partial(
        pl.kernel,
        mesh=mesh,
        compiler_params=pltpu.CompilerParams(needs_layout_passes=False),
        out_type=[
            jax.ShapeDtypeStruct((2 * NP,), jnp.float32),
            jax.ShapeDtypeStruct((2 * PFLAT,), jnp.float32),
        ],
        scratch_types=[
            pltpu.VMEM((INC,), jnp.int32),
            pltpu.VMEM((INC,), jnp.int32),
            pltpu.VMEM((KSUB, SUB), jnp.int32),
            pltpu.VMEM((KSUB, SUB), jnp.float32),
            pltpu.VMEM((KSUB, SUB), jnp.int32),
            pltpu.VMEM((KSUB, SUB), jnp.float32),
            pltpu.VMEM((NP,), jnp.int32),
            pltpu.VMEM((NP,), jnp.float32),
            pltpu.VMEM((2560,), jnp.float32),
            pltpu.VMEM_SHARED((NP,), jnp.float32),
            pltpu.VMEM_SHARED((PFLAT,), jnp.float32),
        ],
    )
    def kern(src_h, dst_h, batch, dinv_out, p_out,
             src_sf, dst_sf, dst_full, val_full, flat_full, ones_full,
             batch_tbl, dinv_tbl, zbuf, deg_sh, p_sh):
        cid = lax.axis_index("c")
        sid = lax.axis_index("s")
        wid = cid * 16 + sid
        z16 = jnp.zeros((16,), jnp.float32)
        o16 = jnp.ones((16,), jnp.float32)

        def zb(i, _):
            zbuf[pl.ds(i * 16, 16)] = z16
            return ()
        lax.fori_loop(0, 160, zb, ())

        def ob(k, _):
            def obi(g, _):
                ones_full[k, pl.ds(g * 16, 16)] = o16
                return ()
            return lax.fori_loop(0, 8, obi, ())
        lax.fori_loop(0, KSUB, ob, ())

        pltpu.sync_copy(batch, batch_tbl)
        pltpu.sync_copy(zbuf.at[pl.ds(0, TECN)],
                        deg_sh.at[pl.ds(sid * TECN, TECN)])
        for q in range(16):
            pltpu.sync_copy(zbuf, p_sh.at[pl.ds(sid * 16 * 2560 + q * 2560, 2560)])
        plsc.subcore_barrier()

        def deg_body(j, _):
            c = sid + 16 * j

            @pl.when(c < NIN)
            def _():
                pltpu.sync_copy(dst_h.at[pl.ds(c * INC, INC)], dst_sf)
                _relayout_1d_to_2d(dst_sf, dst_full)

                def stream_k(k, _):
                    pltpu.sync_copy(ones_full.at[k],
                                    deg_sh.at[dst_full.at[k]], add=True)
                    return ()
                lax.fori_loop(0, KSUB, stream_k, ())
            return ()
        lax.fori_loop(0, 16, deg_body, ())
        plsc.subcore_barrier()

        pltpu.sync_copy(deg_sh, dinv_tbl)

        def dinv_body(i, _):
            d = dinv_tbl[pl.ds(i * 16, 16)] + 1.0
            dinv_tbl[pl.ds(i * 16, 16)] = _rsqrt_newton(d)
            return ()
        lax.fori_loop(0, NP // 16, dinv_body, ())

        def d1_body(i, _):
            zbuf[pl.ds(i * 16, 16)] = dinv_tbl[pl.ds(sid * TECN + i * 16, 16)]
            return ()
        lax.fori_loop(0, TECN // 16, d1_body, ())
        pltpu.sync_copy(zbuf.at[pl.ds(0, TECN)],
                        dinv_out.at[pl.ds(cid * NP + sid * TECN, TECN)])

        def norm_body(j, _):
            c = wid + 32 * j

            @pl.when(c < NIN)
            def _():
                pltpu.sync_copy(src_h.at[pl.ds(c * INC, INC)], src_sf)
                pltpu.sync_copy(dst_h.at[pl.ds(c * INC, INC)], dst_sf)

                def per_k(k, _):
                    def per_g(g, _):
                        sl16 = pl.ds(k * SUB + g * 16, 16)
                        sv = src_sf[sl16]
                        dv = dst_sf[sl16]
                        b = plsc.load_gather(dinv_tbl, [dv])
                        bb = plsc.load_gather(batch_tbl, [dv])
                        sl = pl.ds(g * 16, 16)
                        val_full[k, sl] = b
                        flat_full[k, sl] = bb * NP + sv
                        return ()
                    lax.fori_loop(0, 8, per_g, ())
                    pltpu.sync_copy(val_full.at[k],
                                    p_sh.at[flat_full.at[k]], add=True)
                    return ()
                lax.fori_loop(0, KSUB, per_k, ())
            return ()
        lax.fori_loop(0, 8, norm_body, ())
        plsc.subcore_barrier()

        for q in range(16):
            off = sid * 16 * 2560 + q * 2560
            pltpu.sync_copy(p_sh.at[pl.ds(off, 2560)],
                            p_out.at[pl.ds(cid * PFLAT + off, 2560)])

    return kern


def _mk_aggregate():
    mesh = plsc.VectorSubcoreMesh(core_axis_name="c", subcore_axis_name="s")

    @functools.partial(
        pl.kernel,
        mesh=mesh,
        compiler_params=pltpu.CompilerParams(needs_layout_passes=False),
        out_type=jax.ShapeDtypeStruct((2 * NP, F), jnp.float32),
        scratch_types=[
            pltpu.VMEM((INC,), jnp.int32),
            pltpu.VMEM((INC,), jnp.int32),
            pltpu.VMEM((KSUB, SUB), jnp.int32),
            pltpu.VMEM((SUB, F), jnp.float32),
            pltpu.VMEM((SUB, F), jnp.float32),
            pltpu.SemaphoreType.DMA,
            pltpu.SemaphoreType.DMA,
            pltpu.VMEM_SHARED((NP, F), jnp.float32),
        ],
    )
    def kern(src_h, dst_h, gs_in, e_out,
             src_sf, dst_sf, dst_full, rows, rows1, sem0, sem1, acc_sh):
        cid = lax.axis_index("c")
        sid = lax.axis_index("s")
        wid = cid * 16 + sid
        z16 = jnp.zeros((16,), jnp.float32)

        def zr(r, _):
            def zri(t, _):
                rows[r, pl.ds(t * 16, 16)] = z16
                return ()
            return lax.fori_loop(0, 8, zri, ())
        lax.fori_loop(0, SUB, zr, ())
        for q in range(TECN // SUB):
            pltpu.sync_copy(rows, acc_sh.at[pl.ds(sid * TECN + q * SUB, SUB), :])
        plsc.subcore_barrier()

        def body(j, _):
            c = wid + 32 * j

            @pl.when(c < NIN)
            def _():
                pltpu.sync_copy(src_h.at[pl.ds(c * INC, INC)], src_sf)
                pltpu.sync_copy(dst_h.at[pl.ds(c * INC, INC)], dst_sf)
                _relayout_1d_to_2d(dst_sf, dst_full)

                bufs = (rows, rows1)
                sems = (sem0, sem1)
                pending = pltpu.async_copy(
                    gs_in.at[src_sf.at[pl.ds(0, SUB)]], bufs[0], sems[0])
                for k in range(KSUB):
                    if k + 1 < KSUB:
                        nxt = pltpu.async_copy(
                            gs_in.at[src_sf.at[pl.ds((k + 1) * SUB, SUB)]],
                            bufs[(k + 1) % 2], sems[(k + 1) % 2])
                    pending.wait()
                    pltpu.sync_copy(bufs[k % 2], acc_sh.at[dst_full.at[k]],
                                    add=True)
                    if k + 1 < KSUB:
                        pending = nxt
            return ()
        lax.fori_loop(0, 8, body, ())
        plsc.subcore_barrier()

        pltpu.sync_copy(acc_sh.at[pl.ds(sid * TECN, TECN), :],
                        e_out.at[pl.ds(cid * NP + sid * TECN, TECN), :])

    return kern


def _xw_body(x_ref, w_ref, d1_ref, o_ref):
    o_ref[...] = d1_ref[...] * jnp.dot(x_ref[...], w_ref[...],
                                       preferred_element_type=jnp.float32)


def _mk_xw():
    return pl.pallas_call(
        _xw_body,
        grid=(NP // 1024,),
        in_specs=[
            pl.BlockSpec((1024, F), lambda i: (i, 0)),
            pl.BlockSpec((F, F), lambda i: (0, 0)),
            pl.BlockSpec((1024, 1), lambda i: (i, 0)),
        ],
        out_specs=pl.BlockSpec((1024, F), lambda i: (i, 0)),
        out_shape=jax.ShapeDtypeStruct((NP, F), jnp.float32),
    )


def _final_body(gs_ref, e_ref, d1_ref, bt_ref, p_ref, b1_ref, w2_ref, b2_ref,
                o_ref, accp, accs, accc):
    i = pl.program_id(0)

    @pl.when(i == 0)
    def _():
        accp[...] = jnp.zeros_like(accp)
        accs[...] = jnp.zeros_like(accs)
        accc[...] = jnp.zeros_like(accc)

    gsb = gs_ref[...]
    eb = e_ref[0] + e_ref[1]
    d1 = d1_ref[...]
    hb = jax.nn.relu(d1 * (gsb + eb) + b1_ref[...])
    pb = p_ref[0] + p_ref[1]
    msel = (lax.broadcasted_iota(jnp.int32, (1024, NG), 1)
            == bt_ref[...]).astype(jnp.float32)
    dn = (((0,), (0,)), ((), ()))
    dh = d1 * hb
    accp[...] += jnp.dot(pb, dh, preferred_element_type=jnp.float32)
    accs[...] += lax.dot_general(msel, d1 * dh, dn,
                                 preferred_element_type=jnp.float32)
    accc[...] += lax.dot_general(msel, jnp.ones((1024, F), jnp.float32), dn,
                                 preferred_element_type=jnp.float32)

    @pl.when(i == NP // 1024 - 1)
    def _():
        pooled = (accp[...] + accs[...]) / jnp.maximum(accc[...], 1.0)
        logits = jnp.dot(pooled, w2_ref[...],
                         preferred_element_type=jnp.float32) + b2_ref[...]
        m = jnp.max(logits, axis=1, keepdims=True)
        s = logits - m
        o_ref[...] = s - jnp.log(jnp.sum(jnp.exp(s), axis=1, keepdims=True))


def _mk_final():
    nb = NP // 1024
    return pl.pallas_call(
        _final_body,
        grid=(nb,),
        in_specs=[
            pl.BlockSpec((1024, F), lambda i: (i, 0)),
            pl.BlockSpec((2, 1024, F), lambda i: (0, i, 0)),
            pl.BlockSpec((1024, 1), lambda i: (i, 0)),
            pl.BlockSpec((1024, 1), lambda i: (i, 0)),
            pl.BlockSpec((2, NG, 1024), lambda i: (0, 0, i)),
            pl.BlockSpec((1, F), lambda i: (0, 0)),
            pl.BlockSpec((F, F), lambda i: (0, 0)),
            pl.BlockSpec((1, F), lambda i: (0, 0)),
        ],
        out_specs=pl.BlockSpec((NG, F), lambda i: (0, 0)),
        out_shape=jax.ShapeDtypeStruct((NG, F), jnp.float32),
        scratch_shapes=[
            pltpu.VMEM((NG, F), jnp.float32),
            pltpu.VMEM((NG, F), jnp.float32),
            pltpu.VMEM((NG, F), jnp.float32),
        ],
    )


_edge_stats = _mk_edge_stats()
_aggregate = _mk_aggregate()
_xw = _mk_xw()
_final = _mk_final()


def kernel(x, edge_index, batch, W1, b1, W2, b2):
    src = edge_index[0]
    dst = edge_index[1]
    x_pad = jnp.pad(x, ((0, NP - N), (0, 0)))
    batch_pad = jnp.pad(batch, (0, NP - N), constant_values=NG)

    dinv, p_part = _edge_stats(src, dst, batch_pad)
    d1col = dinv[:NP][:, None]
    gs = _xw(x_pad, W1, d1col)
    e_part = _aggregate(src, dst, gs)

    return _final(gs, e_part.reshape(2, NP, F), d1col, batch_pad[:, None],
                  p_part.reshape(2, NG, NP),
                  b1.reshape(1, F), W2, b2.reshape(1, F))


# speedup vs baseline: 39.7989x; 1.0985x over previous
import functools

import jax
import jax.numpy as jnp
from jax import lax
from jax.experimental import pallas as pl
from jax.experimental.pallas import tpu as pltpu
from jax.experimental.pallas import tpu_sc as plsc

N = 10000
NP = 10240
E = 320000
F = 128
NG = 64
SUB = 128
KSUB = 10
INC = SUB * KSUB
NIN = E // INC
PFLAT = NG * NP
TECN = NP // 16


def _rsqrt_newton(d):
    i = lax.bitcast_convert_type(d, jnp.int32)
    i = 0x5F3759DF - lax.shift_right_logical(i, 1)
    y = lax.bitcast_convert_type(i, jnp.float32)
    for _ in range(3):
        y = y * (1.5 - 0.5 * d * y * y)
    return y


def _relayout_1d_to_2d(buf1, buf2):
    def outer(k, _):
        def inner(g, _):
            buf2[k, pl.ds(g * 16, 16)] = buf1[pl.ds(k * SUB + g * 16, 16)]
            return ()
        return lax.fori_loop(0, SUB // 16, inner, ())
    lax.fori_loop(0, KSUB, outer, ())


def _mk_edge_stats():
    mesh = plsc.VectorSubcoreMesh(core_axis_name="c", subcore_axis_name="s")

    @functools.partial(
        pl.kernel,
        mesh=mesh,
        compiler_params=pltpu.CompilerParams(needs_layout_passes=False),
        out_type=[
            jax.ShapeDtypeStruct((2 * NP,), jnp.float32),
            jax.ShapeDtypeStruct((2 * PFLAT,), jnp.float32),
        ],
        scratch_types=[
            pltpu.VMEM((INC,), jnp.int32),
            pltpu.VMEM((INC,), jnp.int32),
            pltpu.VMEM((INC,), jnp.float32),
            pltpu.VMEM((INC,), jnp.int32),
            pltpu.VMEM((INC,), jnp.float32),
            pltpu.VMEM((NP,), jnp.int32),
            pltpu.VMEM((NP,), jnp.float32),
            pltpu.VMEM((2560,), jnp.float32),
            pltpu.VMEM_SHARED((NP,), jnp.float32),
            pltpu.VMEM_SHARED((PFLAT,), jnp.float32),
        ],
    )
    def kern(src_h, dst_h, batch, dinv_out, p_out,
             src_sf, dst_sf, val_sf, flat_sf, ones_sf,
             batch_tbl, dinv_tbl, zbuf, deg_sh, p_sh):
        cid = lax.axis_index("c")
        sid = lax.axis_index("s")
        wid = cid * 16 + sid
        z16 = jnp.zeros((16,), jnp.float32)
        o16 = jnp.ones((16,), jnp.float32)

        def zb(i, _):
            zbuf[pl.ds(i * 16, 16)] = z16
            return ()
        lax.fori_loop(0, 160, zb, ())

        def ob(i, _):
            ones_sf[pl.ds(i * 16, 16)] = o16
            return ()
        lax.fori_loop(0, INC // 16, ob, ())

        pltpu.sync_copy(batch, batch_tbl)
        pltpu.sync_copy(zbuf.at[pl.ds(0, TECN)],
                        deg_sh.at[pl.ds(sid * TECN, TECN)])
        for q in range(16):
            pltpu.sync_copy(zbuf, p_sh.at[pl.ds(sid * 16 * 2560 + q * 2560, 2560)])
        plsc.subcore_barrier()

        def deg_body(j, _):
            c = sid + 16 * j

            @pl.when(c < NIN)
            def _():
                pltpu.sync_copy(dst_h.at[pl.ds(c * INC, INC)], dst_sf)
                pltpu.sync_copy(ones_sf, deg_sh.at[dst_sf], add=True)
            return ()
        lax.fori_loop(0, 16, deg_body, ())
        plsc.subcore_barrier()

        pltpu.sync_copy(deg_sh, dinv_tbl)

        def dinv_body(i, _):
            d = dinv_tbl[pl.ds(i * 16, 16)] + 1.0
            dinv_tbl[pl.ds(i * 16, 16)] = _rsqrt_newton(d)
            return ()
        lax.fori_loop(0, NP // 16, dinv_body, ())

        def d1_body(i, _):
            zbuf[pl.ds(i * 16, 16)] = dinv_tbl[pl.ds(sid * TECN + i * 16, 16)]
            return ()
        lax.fori_loop(0, TECN // 16, d1_body, ())
        pltpu.sync_copy(zbuf.at[pl.ds(0, TECN)],
                        dinv_out.at[pl.ds(cid * NP + sid * TECN, TECN)])

        def norm_body(j, _):
            c = wid + 32 * j

            @pl.when(c < NIN)
            def _():
                pltpu.sync_copy(src_h.at[pl.ds(c * INC, INC)], src_sf)
                pltpu.sync_copy(dst_h.at[pl.ds(c * INC, INC)], dst_sf)

                def per_g(g, _):
                    sl16 = pl.ds(g * 16, 16)
                    sv = src_sf[sl16]
                    dv = dst_sf[sl16]
                    b = plsc.load_gather(dinv_tbl, [dv])
                    bb = plsc.load_gather(batch_tbl, [dv])
                    val_sf[sl16] = b
                    flat_sf[sl16] = bb * NP + sv
                    return ()
                lax.fori_loop(0, INC // 16, per_g, ())
                pltpu.sync_copy(val_sf, p_sh.at[flat_sf], add=True)
            return ()
        lax.fori_loop(0, 8, norm_body, ())
        plsc.subcore_barrier()

        for q in range(16):
            off = sid * 16 * 2560 + q * 2560
            pltpu.sync_copy(p_sh.at[pl.ds(off, 2560)],
                            p_out.at[pl.ds(cid * PFLAT + off, 2560)])

    return kern


def _mk_aggregate():
    mesh = plsc.VectorSubcoreMesh(core_axis_name="c", subcore_axis_name="s")

    @functools.partial(
        pl.kernel,
        mesh=mesh,
        compiler_params=pltpu.CompilerParams(needs_layout_passes=False),
        out_type=jax.ShapeDtypeStruct((2 * NP, F), jnp.float32),
        scratch_types=[
            pltpu.VMEM((INC,), jnp.int32),
            pltpu.VMEM((INC,), jnp.int32),
            pltpu.VMEM((KSUB, SUB), jnp.int32),
            pltpu.VMEM((SUB, F), jnp.float32),
            pltpu.VMEM((SUB, F), jnp.float32),
            pltpu.SemaphoreType.DMA,
            pltpu.SemaphoreType.DMA,
            pltpu.VMEM_SHARED((NP, F), jnp.float32),
        ],
    )
    def kern(src_h, dst_h, gs_in, e_out,
             src_sf, dst_sf, dst_full, rows, rows1,
             gs0, gs1, acc_sh):
        cid = lax.axis_index("c")
        sid = lax.axis_index("s")
        wid = cid * 16 + sid
        z16 = jnp.zeros((16,), jnp.float32)

        def zr(r, _):
            def zri(t, _):
                rows[r, pl.ds(t * 16, 16)] = z16
                return ()
            return lax.fori_loop(0, 8, zri, ())
        lax.fori_loop(0, SUB, zr, ())
        for q in range(TECN // SUB):
            pltpu.sync_copy(rows, acc_sh.at[pl.ds(sid * TECN + q * SUB, SUB), :])
        plsc.subcore_barrier()

        def body(j, _):
            c = wid + 32 * j

            @pl.when(c < NIN)
            def _():
                pltpu.sync_copy(src_h.at[pl.ds(c * INC, INC)], src_sf)
                pltpu.sync_copy(dst_h.at[pl.ds(c * INC, INC)], dst_sf)
                _relayout_1d_to_2d(dst_sf, dst_full)

                bufs = (rows, rows1)
                gsems = (gs0, gs1)
                pending = pltpu.async_copy(
                    gs_in.at[src_sf.at[pl.ds(0, SUB)]], bufs[0], gsems[0])
                for k in range(KSUB):
                    if k + 1 < KSUB:
                        nxt = pltpu.async_copy(
                            gs_in.at[src_sf.at[pl.ds((k + 1) * SUB, SUB)]],
                            bufs[(k + 1) % 2], gsems[(k + 1) % 2])
                    pending.wait()
                    pltpu.sync_copy(bufs[k % 2], acc_sh.at[dst_full.at[k]],
                                    add=True)
                    if k + 1 < KSUB:
                        pending = nxt
            return ()
        lax.fori_loop(0, 8, body, ())
        plsc.subcore_barrier()

        pltpu.sync_copy(acc_sh.at[pl.ds(sid * TECN, TECN), :],
                        e_out.at[pl.ds(cid * NP + sid * TECN, TECN), :])

    return kern


def _xw_body(x_ref, w_ref, d1_ref, o_ref):
    o_ref[...] = d1_ref[...] * jnp.dot(x_ref[...], w_ref[...],
                                       preferred_element_type=jnp.float32)


def _mk_xw():
    return pl.pallas_call(
        _xw_body,
        grid=(NP // 1024,),
        in_specs=[
            pl.BlockSpec((1024, F), lambda i: (i, 0)),
            pl.BlockSpec((F, F), lambda i: (0, 0)),
            pl.BlockSpec((1024, 1), lambda i: (i, 0)),
        ],
        out_specs=pl.BlockSpec((1024, F), lambda i: (i, 0)),
        out_shape=jax.ShapeDtypeStruct((NP, F), jnp.float32),
    )


def _final_body(gs_ref, e_ref, d1_ref, bt_ref, p_ref, b1_ref, w2_ref, b2_ref,
                o_ref, accp, accs, accc):
    i = pl.program_id(0)

    @pl.when(i == 0)
    def _():
        accp[...] = jnp.zeros_like(accp)
        accs[...] = jnp.zeros_like(accs)
        accc[...] = jnp.zeros_like(accc)

    gsb = gs_ref[...]
    eb = e_ref[0] + e_ref[1]
    d1 = d1_ref[...]
    hb = jax.nn.relu(d1 * (gsb + eb) + b1_ref[...])
    pb = p_ref[0] + p_ref[1]
    msel = (lax.broadcasted_iota(jnp.int32, (1024, NG), 1)
            == bt_ref[...]).astype(jnp.float32)
    dn = (((0,), (0,)), ((), ()))
    dh = d1 * hb
    accp[...] += jnp.dot(pb, dh, preferred_element_type=jnp.float32)
    accs[...] += lax.dot_general(msel, d1 * dh, dn,
                                 preferred_element_type=jnp.float32)
    accc[...] += lax.dot_general(msel, jnp.ones((1024, F), jnp.float32), dn,
                                 preferred_element_type=jnp.float32)

    @pl.when(i == NP // 1024 - 1)
    def _():
        pooled = (accp[...] + accs[...]) / jnp.maximum(accc[...], 1.0)
        logits = jnp.dot(pooled, w2_ref[...],
                         preferred_element_type=jnp.float32) + b2_ref[...]
        m = jnp.max(logits, axis=1, keepdims=True)
        s = logits - m
        o_ref[...] = s - jnp.log(jnp.sum(jnp.exp(s), axis=1, keepdims=True))


def _mk_final():
    nb = NP // 1024
    return pl.pallas_call(
        _final_body,
        grid=(nb,),
        in_specs=[
            pl.BlockSpec((1024, F), lambda i: (i, 0)),
            pl.BlockSpec((2, 1024, F), lambda i: (0, i, 0)),
            pl.BlockSpec((1024, 1), lambda i: (i, 0)),
            pl.BlockSpec((1024, 1), lambda i: (i, 0)),
            pl.BlockSpec((2, NG, 1024), lambda i: (0, 0, i)),
            pl.BlockSpec((1, F), lambda i: (0, 0)),
            pl.BlockSpec((F, F), lambda i: (0, 0)),
            pl.BlockSpec((1, F), lambda i: (0, 0)),
        ],
        out_specs=pl.BlockSpec((NG, F), lambda i: (0, 0)),
        out_shape=jax.ShapeDtypeStruct((NG, F), jnp.float32),
        scratch_shapes=[
            pltpu.VMEM((NG, F), jnp.float32),
            pltpu.VMEM((NG, F), jnp.float32),
            pltpu.VMEM((NG, F), jnp.float32),
        ],
    )


_edge_stats = _mk_edge_stats()
_aggregate = _mk_aggregate()
_xw = _mk_xw()
_final = _mk_final()


def kernel(x, edge_index, batch, W1, b1, W2, b2):
    src = edge_index[0]
    dst = edge_index[1]
    x_pad = jnp.pad(x, ((0, NP - N), (0, 0)))
    batch_pad = jnp.pad(batch, (0, NP - N), constant_values=NG)

    dinv, p_part = _edge_stats(src, dst, batch_pad)
    d1col = dinv[:NP][:, None]
    gs = _xw(x_pad, W1, d1col)
    e_part = _aggregate(src, dst, gs)

    return _final(gs, e_part.reshape(2, NP, F), d1col, batch_pad[:, None],
                  p_part.reshape(2, NG, NP),
                  b1.reshape(1, F), W2, b2.reshape(1, F))


# speedup vs baseline: 44.0560x; 1.1070x over previous
import functools

import jax
import jax.numpy as jnp
from jax import lax
from jax.experimental import pallas as pl
from jax.experimental.pallas import tpu as pltpu
from jax.experimental.pallas import tpu_sc as plsc

N = 10000
NP = 10240
E = 320000
F = 128
NG = 64
SUB = 128
KSUB = 10
INC = SUB * KSUB
NIN = E // INC
PFLAT = NG * NP
TECN = NP // 16


def _rsqrt_newton(d):
    i = lax.bitcast_convert_type(d, jnp.int32)
    i = 0x5F3759DF - lax.shift_right_logical(i, 1)
    y = lax.bitcast_convert_type(i, jnp.float32)
    for _ in range(3):
        y = y * (1.5 - 0.5 * d * y * y)
    return y


def _relayout_1d_to_2d(buf1, buf2):
    def outer(k, _):
        def inner(g, _):
            buf2[k, pl.ds(g * 16, 16)] = buf1[pl.ds(k * SUB + g * 16, 16)]
            return ()
        return lax.fori_loop(0, SUB // 16, inner, ())
    lax.fori_loop(0, KSUB, outer, ())


def _mk_edge_stats():
    mesh = plsc.VectorSubcoreMesh(core_axis_name="c", subcore_axis_name="s")

    @functools.partial(
        pl.kernel,
        mesh=mesh,
        compiler_params=pltpu.CompilerParams(needs_layout_passes=False),
        out_type=[
            jax.ShapeDtypeStruct((2 * NP,), jnp.float32),
            jax.ShapeDtypeStruct((2 * PFLAT,), jnp.float32),
        ],
        scratch_types=[
            pltpu.VMEM((E // 32,), jnp.int32),
            pltpu.VMEM((E // 32,), jnp.int32),
            pltpu.VMEM((E // 32,), jnp.float32),
            pltpu.VMEM((E // 32,), jnp.int32),
            pltpu.VMEM((NP,), jnp.int32),
            pltpu.VMEM((NP,), jnp.float32),
            pltpu.VMEM((2560,), jnp.float32),
            pltpu.VMEM_SHARED((NP,), jnp.float32),
            pltpu.VMEM_SHARED((PFLAT,), jnp.float32),
        ],
    )
    def kern(eif, batch, dinv_out, p_out,
             src_sf, dst_sf, val_sf, flat_sf,
             batch_tbl, dinv_tbl, zbuf, deg_sh, p_sh):
        cid = lax.axis_index("c")
        sid = lax.axis_index("s")
        wid = cid * 16 + sid
        z16 = jnp.zeros((16,), jnp.float32)
        o16 = jnp.ones((16,), jnp.float32)
        ENRM = E // 32

        def zb(i, _):
            zbuf[pl.ds(i * 16, 16)] = z16
            return ()
        lax.fori_loop(0, 160, zb, ())

        def ob(i, _):
            val_sf[pl.ds(i * 16, 16)] = o16
            return ()
        lax.fori_loop(0, ENRM // 16, ob, ())

        pltpu.sync_copy(batch, batch_tbl)
        pltpu.sync_copy(zbuf.at[pl.ds(0, TECN)],
                        deg_sh.at[pl.ds(sid * TECN, TECN)])
        for q in range(16):
            pltpu.sync_copy(zbuf, p_sh.at[pl.ds(sid * 16 * 2560 + q * 2560, 2560)])
        plsc.subcore_barrier()

        for q in range(2):
            pltpu.sync_copy(
                eif.at[pl.ds(E + sid * 2 * ENRM + q * ENRM, ENRM)], dst_sf)
            pltpu.sync_copy(val_sf, deg_sh.at[dst_sf], add=True)
        plsc.subcore_barrier()

        pltpu.sync_copy(deg_sh, dinv_tbl)

        def dinv_body(i, _):
            d = dinv_tbl[pl.ds(i * 16, 16)] + 1.0
            dinv_tbl[pl.ds(i * 16, 16)] = _rsqrt_newton(d)
            return ()
        lax.fori_loop(0, NP // 16, dinv_body, ())

        def d1_body(i, _):
            zbuf[pl.ds(i * 16, 16)] = dinv_tbl[pl.ds(sid * TECN + i * 16, 16)]
            return ()
        lax.fori_loop(0, TECN // 16, d1_body, ())
        pltpu.sync_copy(zbuf.at[pl.ds(0, TECN)],
                        dinv_out.at[pl.ds(cid * NP + sid * TECN, TECN)])

        pltpu.sync_copy(eif.at[pl.ds(wid * ENRM, ENRM)], src_sf)
        pltpu.sync_copy(eif.at[pl.ds(E + wid * ENRM, ENRM)],
                        dst_sf.at[pl.ds(0, ENRM)])

        def per_g(g, _):
            sl16 = pl.ds(g * 16, 16)
            sv = src_sf[sl16]
            dv = dst_sf[sl16]
            b = plsc.load_gather(dinv_tbl, [dv])
            bb = plsc.load_gather(batch_tbl, [dv])
            val_sf[sl16] = b
            flat_sf[sl16] = bb * NP + sv
            return ()
        lax.fori_loop(0, ENRM // 16, per_g, ())
        pltpu.sync_copy(val_sf, p_sh.at[flat_sf], add=True)
        plsc.subcore_barrier()

        for q in range(16):
            off = sid * 16 * 2560 + q * 2560
            pltpu.sync_copy(p_sh.at[pl.ds(off, 2560)],
                            p_out.at[pl.ds(cid * PFLAT + off, 2560)])

    return kern


def _mk_aggregate():
    mesh = plsc.VectorSubcoreMesh(core_axis_name="c", subcore_axis_name="s")

    @functools.partial(
        pl.kernel,
        mesh=mesh,
        compiler_params=pltpu.CompilerParams(needs_layout_passes=False),
        out_type=jax.ShapeDtypeStruct((2 * NP, F), jnp.float32),
        scratch_types=[
            pltpu.VMEM((INC,), jnp.int32),
            pltpu.VMEM((INC,), jnp.int32),
            pltpu.VMEM((KSUB, SUB), jnp.int32),
            pltpu.VMEM((SUB, F), jnp.float32),
            pltpu.VMEM((SUB, F), jnp.float32),
            pltpu.SemaphoreType.DMA,
            pltpu.SemaphoreType.DMA,
            pltpu.VMEM_SHARED((NP, F), jnp.float32),
        ],
    )
    def kern(eif, gs_in, e_out,
             src_sf, dst_sf, dst_full, rows, rows1,
             gs0, gs1, acc_sh):
        cid = lax.axis_index("c")
        sid = lax.axis_index("s")
        wid = cid * 16 + sid
        z16 = jnp.zeros((16,), jnp.float32)

        def zr(r, _):
            def zri(t, _):
                rows[r, pl.ds(t * 16, 16)] = z16
                return ()
            return lax.fori_loop(0, 8, zri, ())
        lax.fori_loop(0, SUB, zr, ())
        for q in range(TECN // SUB):
            pltpu.sync_copy(rows, acc_sh.at[pl.ds(sid * TECN + q * SUB, SUB), :])
        plsc.subcore_barrier()

        def body(j, _):
            c = wid + 32 * j

            @pl.when(c < NIN)
            def _():
                pltpu.sync_copy(eif.at[pl.ds(c * INC, INC)], src_sf)
                pltpu.sync_copy(eif.at[pl.ds(E + c * INC, INC)], dst_sf)
                _relayout_1d_to_2d(dst_sf, dst_full)

                bufs = (rows, rows1)
                gsems = (gs0, gs1)
                pending = pltpu.async_copy(
                    gs_in.at[src_sf.at[pl.ds(0, SUB)]], bufs[0], gsems[0])
                for k in range(KSUB):
                    if k + 1 < KSUB:
                        nxt = pltpu.async_copy(
                            gs_in.at[src_sf.at[pl.ds((k + 1) * SUB, SUB)]],
                            bufs[(k + 1) % 2], gsems[(k + 1) % 2])
                    pending.wait()
                    pltpu.sync_copy(bufs[k % 2], acc_sh.at[dst_full.at[k]],
                                    add=True)
                    if k + 1 < KSUB:
                        pending = nxt
            return ()
        lax.fori_loop(0, 8, body, ())
        plsc.subcore_barrier()

        pltpu.sync_copy(acc_sh.at[pl.ds(sid * TECN, TECN), :],
                        e_out.at[pl.ds(cid * NP + sid * TECN, TECN), :])

    return kern


def _xw_body(x_ref, w_ref, d1_ref, o_ref):
    o_ref[...] = d1_ref[...] * jnp.dot(x_ref[...], w_ref[...],
                                       preferred_element_type=jnp.float32)


def _mk_xw():
    return pl.pallas_call(
        _xw_body,
        grid=(NP // 1024,),
        in_specs=[
            pl.BlockSpec((1024, F), lambda i: (i, 0)),
            pl.BlockSpec((F, F), lambda i: (0, 0)),
            pl.BlockSpec((1024, 1), lambda i: (i, 0)),
        ],
        out_specs=pl.BlockSpec((1024, F), lambda i: (i, 0)),
        out_shape=jax.ShapeDtypeStruct((NP, F), jnp.float32),
    )


def _final_body(gs_ref, e_ref, d1_ref, bt_ref, p_ref, b1_ref, w2_ref, b2_ref,
                o_ref, accp, accs, accc):
    i = pl.program_id(0)

    @pl.when(i == 0)
    def _():
        accp[...] = jnp.zeros_like(accp)
        accs[...] = jnp.zeros_like(accs)
        accc[...] = jnp.zeros_like(accc)

    gsb = gs_ref[...]
    eb = e_ref[0] + e_ref[1]
    d1 = d1_ref[...]
    hb = jax.nn.relu(d1 * (gsb + eb) + b1_ref[...])
    pb = p_ref[0] + p_ref[1]
    msel = (lax.broadcasted_iota(jnp.int32, (1024, NG), 1)
            == bt_ref[...]).astype(jnp.float32)
    dn = (((0,), (0,)), ((), ()))
    dh = d1 * hb
    accp[...] += jnp.dot(pb, dh, preferred_element_type=jnp.float32)
    accs[...] += lax.dot_general(msel, d1 * dh, dn,
                                 preferred_element_type=jnp.float32)
    accc[...] += lax.dot_general(msel, jnp.ones((1024, F), jnp.float32), dn,
                                 preferred_element_type=jnp.float32)

    @pl.when(i == NP // 1024 - 1)
    def _():
        pooled = (accp[...] + accs[...]) / jnp.maximum(accc[...], 1.0)
        logits = jnp.dot(pooled, w2_ref[...],
                         preferred_element_type=jnp.float32) + b2_ref[...]
        m = jnp.max(logits, axis=1, keepdims=True)
        s = logits - m
        o_ref[...] = s - jnp.log(jnp.sum(jnp.exp(s), axis=1, keepdims=True))


def _mk_final():
    nb = NP // 1024
    return pl.pallas_call(
        _final_body,
        grid=(nb,),
        in_specs=[
            pl.BlockSpec((1024, F), lambda i: (i, 0)),
            pl.BlockSpec((2, 1024, F), lambda i: (0, i, 0)),
            pl.BlockSpec((1024, 1), lambda i: (i, 0)),
            pl.BlockSpec((1024, 1), lambda i: (i, 0)),
            pl.BlockSpec((2, NG, 1024), lambda i: (0, 0, i)),
            pl.BlockSpec((1, F), lambda i: (0, 0)),
            pl.BlockSpec((F, F), lambda i: (0, 0)),
            pl.BlockSpec((1, F), lambda i: (0, 0)),
        ],
        out_specs=pl.BlockSpec((NG, F), lambda i: (0, 0)),
        out_shape=jax.ShapeDtypeStruct((NG, F), jnp.float32),
        scratch_shapes=[
            pltpu.VMEM((NG, F), jnp.float32),
            pltpu.VMEM((NG, F), jnp.float32),
            pltpu.VMEM((NG, F), jnp.float32),
        ],
    )


_edge_stats = _mk_edge_stats()
_aggregate = _mk_aggregate()
_xw = _mk_xw()
_final = _mk_final()


def kernel(x, edge_index, batch, W1, b1, W2, b2):
    eif = edge_index.reshape(2 * E)
    x_pad = jnp.pad(x, ((0, NP - N), (0, 0)))
    batch_pad = jnp.pad(batch, (0, NP - N), constant_values=NG)

    dinv, p_part = _edge_stats(eif, batch_pad)
    d1col = dinv[:NP][:, None]
    gs = _xw(x_pad, W1, d1col)
    e_part = _aggregate(eif, gs)

    return _final(gs, e_part.reshape(2, NP, F), d1col, batch_pad[:, None],
                  p_part.reshape(2, NG, NP),
                  b1.reshape(1, F), W2, b2.reshape(1, F))


# speedup vs baseline: 45.7792x; 1.0391x over previous
import functools

import jax
import jax.numpy as jnp
from jax import lax
from jax.experimental import pallas as pl
from jax.experimental.pallas import tpu as pltpu
from jax.experimental.pallas import tpu_sc as plsc

N = 10000
NP = 10240
E = 320000
F = 128
NG = 64
SUB = 128
KSUB = 10
INC = SUB * KSUB
NIN = E // INC
PFLAT = NG * NP
TECN = NP // 16


def _rsqrt_newton(d):
    i = lax.bitcast_convert_type(d, jnp.int32)
    i = 0x5F3759DF - lax.shift_right_logical(i, 1)
    y = lax.bitcast_convert_type(i, jnp.float32)
    for _ in range(3):
        y = y * (1.5 - 0.5 * d * y * y)
    return y


def _relayout_1d_to_2d(buf1, buf2):
    def outer(k, _):
        def inner(g, _):
            buf2[k, pl.ds(g * 16, 16)] = buf1[pl.ds(k * SUB + g * 16, 16)]
            return ()
        return lax.fori_loop(0, SUB // 16, inner, ())
    lax.fori_loop(0, KSUB, outer, ())


def _mk_edge_stats():
    mesh = plsc.VectorSubcoreMesh(core_axis_name="c", subcore_axis_name="s")

    @functools.partial(
        pl.kernel,
        mesh=mesh,
        compiler_params=pltpu.CompilerParams(needs_layout_passes=False),
        out_type=[
            jax.ShapeDtypeStruct((2 * NP,), jnp.float32),
            jax.ShapeDtypeStruct((2 * PFLAT,), jnp.float32),
        ],
        scratch_types=[
            pltpu.VMEM((E // 32,), jnp.int32),
            pltpu.VMEM((E // 32,), jnp.int32),
            pltpu.VMEM((E // 32,), jnp.float32),
            pltpu.VMEM((E // 32,), jnp.int32),
            pltpu.VMEM((NP,), jnp.int32),
            pltpu.VMEM((NP,), jnp.float32),
            pltpu.VMEM((2560,), jnp.float32),
            pltpu.VMEM_SHARED((NP,), jnp.float32),
            pltpu.VMEM_SHARED((PFLAT,), jnp.float32),
        ],
    )
    def kern(eif, batch, dinv_out, p_out,
             src_sf, dst_sf, val_sf, flat_sf,
             batch_tbl, dinv_tbl, zbuf, deg_sh, p_sh):
        cid = lax.axis_index("c")
        sid = lax.axis_index("s")
        wid = cid * 16 + sid
        z16 = jnp.zeros((16,), jnp.float32)
        o16 = jnp.ones((16,), jnp.float32)
        ENRM = E // 32

        def zb(i, _):
            zbuf[pl.ds(i * 16, 16)] = z16
            return ()
        lax.fori_loop(0, 160, zb, ())

        def ob(i, _):
            val_sf[pl.ds(i * 16, 16)] = o16
            return ()
        lax.fori_loop(0, ENRM // 16, ob, ())

        pltpu.sync_copy(batch, batch_tbl)
        pltpu.sync_copy(zbuf.at[pl.ds(0, TECN)],
                        deg_sh.at[pl.ds(sid * TECN, TECN)])
        for q in range(16):
            pltpu.sync_copy(zbuf, p_sh.at[pl.ds(sid * 16 * 2560 + q * 2560, 2560)])
        plsc.subcore_barrier()

        for q in range(2):
            pltpu.sync_copy(
                eif.at[pl.ds(E + sid * 2 * ENRM + q * ENRM, ENRM)], dst_sf)
            pltpu.sync_copy(val_sf, deg_sh.at[dst_sf], add=True)
        plsc.subcore_barrier()

        pltpu.sync_copy(deg_sh, dinv_tbl)

        def dinv_body(i, _):
            d = dinv_tbl[pl.ds(i * 16, 16)] + 1.0
            dinv_tbl[pl.ds(i * 16, 16)] = _rsqrt_newton(d)
            return ()
        lax.fori_loop(0, NP // 16, dinv_body, ())

        def d1_body(i, _):
            zbuf[pl.ds(i * 16, 16)] = dinv_tbl[pl.ds(sid * TECN + i * 16, 16)]
            return ()
        lax.fori_loop(0, TECN // 16, d1_body, ())
        pltpu.sync_copy(zbuf.at[pl.ds(0, TECN)],
                        dinv_out.at[pl.ds(cid * NP + sid * TECN, TECN)])

        pltpu.sync_copy(eif.at[pl.ds(wid * ENRM, ENRM)], src_sf)
        pltpu.sync_copy(eif.at[pl.ds(E + wid * ENRM, ENRM)],
                        dst_sf.at[pl.ds(0, ENRM)])

        def per_g(g, _):
            sl16 = pl.ds(g * 16, 16)
            sv = src_sf[sl16]
            dv = dst_sf[sl16]
            b = plsc.load_gather(dinv_tbl, [dv])
            bb = plsc.load_gather(batch_tbl, [dv])
            val_sf[sl16] = b
            flat_sf[sl16] = bb * NP + sv
            return ()
        lax.fori_loop(0, ENRM // 16, per_g, ())
        pltpu.sync_copy(val_sf, p_sh.at[flat_sf], add=True)
        plsc.subcore_barrier()

        for q in range(16):
            off = sid * 16 * 2560 + q * 2560
            pltpu.sync_copy(p_sh.at[pl.ds(off, 2560)],
                            p_out.at[pl.ds(cid * PFLAT + off, 2560)])

    return kern


def _mk_aggregate():
    mesh = plsc.VectorSubcoreMesh(core_axis_name="c", subcore_axis_name="s")

    @functools.partial(
        pl.kernel,
        mesh=mesh,
        compiler_params=pltpu.CompilerParams(needs_layout_passes=False),
        out_type=jax.ShapeDtypeStruct((2 * NP, F), jnp.float32),
        scratch_types=[
            pltpu.VMEM((E // 32,), jnp.int32),
            pltpu.VMEM((E // 32,), jnp.int32),
            pltpu.VMEM((80, F), jnp.float32),
            pltpu.VMEM((80, F), jnp.float32),
            pltpu.SemaphoreType.DMA,
            pltpu.SemaphoreType.DMA,
            pltpu.VMEM_SHARED((NP, F), jnp.float32),
        ],
    )
    def kern(eif, gs_in, e_out,
             src_sf, dst_sf, rows, rows1,
             gs0, gs1, acc_sh):
        cid = lax.axis_index("c")
        sid = lax.axis_index("s")
        wid = cid * 16 + sid
        z16 = jnp.zeros((16,), jnp.float32)
        EAGG = E // 32
        SUBC = 80
        NK = EAGG // SUBC

        def zr(r, _):
            def zri(t, _):
                rows[r, pl.ds(t * 16, 16)] = z16
                return ()
            return lax.fori_loop(0, 8, zri, ())
        lax.fori_loop(0, SUBC, zr, ())
        for q in range(TECN // SUBC):
            pltpu.sync_copy(
                rows, acc_sh.at[pl.ds(sid * TECN + q * SUBC, SUBC), :])
        plsc.subcore_barrier()

        pltpu.sync_copy(eif.at[pl.ds(wid * EAGG, EAGG)], src_sf)
        pltpu.sync_copy(eif.at[pl.ds(E + wid * EAGG, EAGG)], dst_sf)

        def drain(buf, sem):
            pltpu.make_async_copy(gs_in.at[pl.ds(0, SUBC), :], buf, sem).wait()

        pltpu.async_copy(gs_in.at[src_sf.at[pl.ds(0, SUBC)]], rows, gs0)

        def body2(j, _):
            k0 = j * 2
            pltpu.async_copy(
                gs_in.at[src_sf.at[pl.ds((k0 + 1) * SUBC, SUBC)]], rows1, gs1)
            drain(rows, gs0)
            pltpu.sync_copy(rows, acc_sh.at[dst_sf.at[pl.ds(k0 * SUBC, SUBC)]],
                            add=True)

            @pl.when(k0 + 2 < NK)
            def _():
                pltpu.async_copy(
                    gs_in.at[src_sf.at[pl.ds((k0 + 2) * SUBC, SUBC)]],
                    rows, gs0)
            drain(rows1, gs1)
            pltpu.sync_copy(rows1,
                            acc_sh.at[dst_sf.at[pl.ds((k0 + 1) * SUBC, SUBC)]],
                            add=True)
            return ()
        lax.fori_loop(0, (NK - 1) // 2, body2, ())
        drain(rows, gs0)
        pltpu.sync_copy(rows, acc_sh.at[dst_sf.at[pl.ds((NK - 1) * SUBC, SUBC)]],
                        add=True)
        plsc.subcore_barrier()

        pltpu.sync_copy(acc_sh.at[pl.ds(sid * TECN, TECN), :],
                        e_out.at[pl.ds(cid * NP + sid * TECN, TECN), :])

    return kern


def _xw_body(x_ref, w_ref, d1_ref, o_ref):
    o_ref[...] = d1_ref[...] * jnp.dot(x_ref[...], w_ref[...],
                                       preferred_element_type=jnp.float32)


def _mk_xw():
    return pl.pallas_call(
        _xw_body,
        grid=(NP // 1024,),
        in_specs=[
            pl.BlockSpec((1024, F), lambda i: (i, 0)),
            pl.BlockSpec((F, F), lambda i: (0, 0)),
            pl.BlockSpec((1024, 1), lambda i: (i, 0)),
        ],
        out_specs=pl.BlockSpec((1024, F), lambda i: (i, 0)),
        out_shape=jax.ShapeDtypeStruct((NP, F), jnp.float32),
    )


def _final_body(gs_ref, e_ref, d1_ref, bt_ref, p_ref, b1_ref, w2_ref, b2_ref,
                o_ref, accp, accs, accc):
    i = pl.program_id(0)

    @pl.when(i == 0)
    def _():
        accp[...] = jnp.zeros_like(accp)
        accs[...] = jnp.zeros_like(accs)
        accc[...] = jnp.zeros_like(accc)

    gsb = gs_ref[...]
    eb = e_ref[0] + e_ref[1]
    d1 = d1_ref[...]
    hb = jax.nn.relu(d1 * (gsb + eb) + b1_ref[...])
    pb = p_ref[0] + p_ref[1]
    msel = (lax.broadcasted_iota(jnp.int32, (1024, NG), 1)
            == bt_ref[...]).astype(jnp.float32)
    dn = (((0,), (0,)), ((), ()))
    dh = d1 * hb
    accp[...] += jnp.dot(pb, dh, preferred_element_type=jnp.float32)
    accs[...] += lax.dot_general(msel, d1 * dh, dn,
                                 preferred_element_type=jnp.float32)
    accc[...] += lax.dot_general(msel, jnp.ones((1024, F), jnp.float32), dn,
                                 preferred_element_type=jnp.float32)

    @pl.when(i == NP // 1024 - 1)
    def _():
        pooled = (accp[...] + accs[...]) / jnp.maximum(accc[...], 1.0)
        logits = jnp.dot(pooled, w2_ref[...],
                         preferred_element_type=jnp.float32) + b2_ref[...]
        m = jnp.max(logits, axis=1, keepdims=True)
        s = logits - m
        o_ref[...] = s - jnp.log(jnp.sum(jnp.exp(s), axis=1, keepdims=True))


def _mk_final():
    nb = NP // 1024
    return pl.pallas_call(
        _final_body,
        grid=(nb,),
        in_specs=[
            pl.BlockSpec((1024, F), lambda i: (i, 0)),
            pl.BlockSpec((2, 1024, F), lambda i: (0, i, 0)),
            pl.BlockSpec((1024, 1), lambda i: (i, 0)),
            pl.BlockSpec((1024, 1), lambda i: (i, 0)),
            pl.BlockSpec((2, NG, 1024), lambda i: (0, 0, i)),
            pl.BlockSpec((1, F), lambda i: (0, 0)),
            pl.BlockSpec((F, F), lambda i: (0, 0)),
            pl.BlockSpec((1, F), lambda i: (0, 0)),
        ],
        out_specs=pl.BlockSpec((NG, F), lambda i: (0, 0)),
        out_shape=jax.ShapeDtypeStruct((NG, F), jnp.float32),
        scratch_shapes=[
            pltpu.VMEM((NG, F), jnp.float32),
            pltpu.VMEM((NG, F), jnp.float32),
            pltpu.VMEM((NG, F), jnp.float32),
        ],
    )


_edge_stats = _mk_edge_stats()
_aggregate = _mk_aggregate()
_xw = _mk_xw()
_final = _mk_final()


def kernel(x, edge_index, batch, W1, b1, W2, b2):
    eif = edge_index.reshape(2 * E)
    x_pad = jnp.pad(x, ((0, NP - N), (0, 0)))
    batch_pad = jnp.pad(batch, (0, NP - N), constant_values=NG)

    dinv, p_part = _edge_stats(eif, batch_pad)
    d1col = dinv[:NP][:, None]
    gs = _xw(x_pad, W1, d1col)
    e_part = _aggregate(eif, gs)

    return _final(gs, e_part.reshape(2, NP, F), d1col, batch_pad[:, None],
                  p_part.reshape(2, NG, NP),
                  b1.reshape(1, F), W2, b2.reshape(1, F))


# speedup vs baseline: 46.7946x; 1.0222x over previous
import functools

import jax
import jax.numpy as jnp
from jax import lax
from jax.experimental import pallas as pl
from jax.experimental.pallas import tpu as pltpu
from jax.experimental.pallas import tpu_sc as plsc

N = 10000
NP = 10240
E = 320000
F = 128
NG = 64
SUB = 128
KSUB = 10
INC = SUB * KSUB
NIN = E // INC
PFLAT = NG * NP
TECN = NP // 16


def _rsqrt_newton(d):
    i = lax.bitcast_convert_type(d, jnp.int32)
    i = 0x5F3759DF - lax.shift_right_logical(i, 1)
    y = lax.bitcast_convert_type(i, jnp.float32)
    for _ in range(3):
        y = y * (1.5 - 0.5 * d * y * y)
    return y


def _relayout_1d_to_2d(buf1, buf2):
    def outer(k, _):
        def inner(g, _):
            buf2[k, pl.ds(g * 16, 16)] = buf1[pl.ds(k * SUB + g * 16, 16)]
            return ()
        return lax.fori_loop(0, SUB // 16, inner, ())
    lax.fori_loop(0, KSUB, outer, ())


def _mk_edge_stats():
    mesh = plsc.VectorSubcoreMesh(core_axis_name="c", subcore_axis_name="s")

    @functools.partial(
        pl.kernel,
        mesh=mesh,
        compiler_params=pltpu.CompilerParams(needs_layout_passes=False),
        out_type=[
            jax.ShapeDtypeStruct((2 * NP,), jnp.float32),
            jax.ShapeDtypeStruct((2 * PFLAT,), jnp.float32),
        ],
        scratch_types=[
            pltpu.VMEM((E // 32,), jnp.int32),
            pltpu.VMEM((E // 32,), jnp.int32),
            pltpu.VMEM((E // 32,), jnp.float32),
            pltpu.VMEM((E // 32,), jnp.int32),
            pltpu.VMEM((E // 32,), jnp.int32),
            pltpu.VMEM((NP,), jnp.int32),
            pltpu.VMEM((NP,), jnp.float32),
            pltpu.VMEM((2560,), jnp.float32),
            pltpu.SemaphoreType.DMA,
            pltpu.SemaphoreType.DMA,
            pltpu.SemaphoreType.DMA,
            pltpu.VMEM_SHARED((NP,), jnp.float32),
            pltpu.VMEM_SHARED((PFLAT,), jnp.float32),
        ],
    )
    def kern(eif, batch, dinv_out, p_out,
             src_sf, dst_sf, val_sf, flat_sf, dst2_sf,
             batch_tbl, dinv_tbl, zbuf, sm0, sm1, sm2, deg_sh, p_sh):
        cid = lax.axis_index("c")
        sid = lax.axis_index("s")
        wid = cid * 16 + sid
        z16 = jnp.zeros((16,), jnp.float32)
        o16 = jnp.ones((16,), jnp.float32)
        ENRM = E // 32

        h0 = pltpu.async_copy(
            eif.at[pl.ds(E + sid * 2 * ENRM, ENRM)], dst_sf, sm0)
        h1 = pltpu.async_copy(
            eif.at[pl.ds(E + sid * 2 * ENRM + ENRM, ENRM)], dst2_sf, sm1)
        h2 = pltpu.async_copy(eif.at[pl.ds(wid * ENRM, ENRM)], src_sf, sm2)

        def zb(i, _):
            zbuf[pl.ds(i * 16, 16)] = z16
            return ()
        lax.fori_loop(0, 160, zb, ())

        def ob(i, _):
            val_sf[pl.ds(i * 16, 16)] = o16
            return ()
        lax.fori_loop(0, ENRM // 16, ob, ())

        pltpu.sync_copy(batch, batch_tbl)
        pltpu.sync_copy(zbuf.at[pl.ds(0, TECN)],
                        deg_sh.at[pl.ds(sid * TECN, TECN)])
        for q in range(16):
            pltpu.sync_copy(zbuf, p_sh.at[pl.ds(sid * 16 * 2560 + q * 2560, 2560)])
        plsc.subcore_barrier()

        h0.wait()
        pltpu.sync_copy(val_sf, deg_sh.at[dst_sf], add=True)
        h1.wait()
        h3 = pltpu.async_copy(eif.at[pl.ds(E + wid * ENRM, ENRM)], dst_sf, sm0)
        pltpu.sync_copy(val_sf, deg_sh.at[dst2_sf], add=True)
        plsc.subcore_barrier()

        pltpu.sync_copy(deg_sh, dinv_tbl)

        def dinv_body(i, _):
            d = dinv_tbl[pl.ds(i * 16, 16)] + 1.0
            dinv_tbl[pl.ds(i * 16, 16)] = _rsqrt_newton(d)
            return ()
        lax.fori_loop(0, NP // 16, dinv_body, ())

        def d1_body(i, _):
            zbuf[pl.ds(i * 16, 16)] = dinv_tbl[pl.ds(sid * TECN + i * 16, 16)]
            return ()
        lax.fori_loop(0, TECN // 16, d1_body, ())
        pltpu.sync_copy(zbuf.at[pl.ds(0, TECN)],
                        dinv_out.at[pl.ds(cid * NP + sid * TECN, TECN)])

        h2.wait()
        h3.wait()

        def per_g(g, _):
            sl16 = pl.ds(g * 16, 16)
            sv = src_sf[sl16]
            dv = dst_sf[sl16]
            b = plsc.load_gather(dinv_tbl, [dv])
            bb = plsc.load_gather(batch_tbl, [dv])
            val_sf[sl16] = b
            flat_sf[sl16] = bb * NP + sv
            return ()
        lax.fori_loop(0, ENRM // 16, per_g, ())
        pltpu.sync_copy(val_sf, p_sh.at[flat_sf], add=True)
        plsc.subcore_barrier()

        for q in range(16):
            off = sid * 16 * 2560 + q * 2560
            pltpu.sync_copy(p_sh.at[pl.ds(off, 2560)],
                            p_out.at[pl.ds(cid * PFLAT + off, 2560)])

    return kern


def _mk_aggregate():
    mesh = plsc.VectorSubcoreMesh(core_axis_name="c", subcore_axis_name="s")

    @functools.partial(
        pl.kernel,
        mesh=mesh,
        compiler_params=pltpu.CompilerParams(needs_layout_passes=False),
        out_type=jax.ShapeDtypeStruct((2 * NP, F), jnp.float32),
        scratch_types=[
            pltpu.VMEM((E // 32,), jnp.int32),
            pltpu.VMEM((E // 32,), jnp.int32),
            pltpu.VMEM((80, F), jnp.float32),
            pltpu.VMEM((80, F), jnp.float32),
            pltpu.SemaphoreType.DMA,
            pltpu.SemaphoreType.DMA,
            pltpu.VMEM_SHARED((NP, F), jnp.float32),
        ],
    )
    def kern(eif, gs_in, e_out,
             src_sf, dst_sf, rows, rows1,
             gs0, gs1, acc_sh):
        cid = lax.axis_index("c")
        sid = lax.axis_index("s")
        wid = cid * 16 + sid
        z16 = jnp.zeros((16,), jnp.float32)
        EAGG = E // 32
        SUBC = 80
        NK = EAGG // SUBC

        def zr(r, _):
            def zri(t, _):
                rows[r, pl.ds(t * 16, 16)] = z16
                return ()
            return lax.fori_loop(0, 8, zri, ())
        lax.fori_loop(0, SUBC, zr, ())
        for q in range(TECN // SUBC):
            pltpu.sync_copy(
                rows, acc_sh.at[pl.ds(sid * TECN + q * SUBC, SUBC), :])
        plsc.subcore_barrier()

        pltpu.sync_copy(eif.at[pl.ds(wid * EAGG, EAGG)], src_sf)
        pltpu.sync_copy(eif.at[pl.ds(E + wid * EAGG, EAGG)], dst_sf)

        def drain(buf, sem):
            pltpu.make_async_copy(gs_in.at[pl.ds(0, SUBC), :], buf, sem).wait()

        pltpu.async_copy(gs_in.at[src_sf.at[pl.ds(0, SUBC)]], rows, gs0)

        def body2(j, _):
            k0 = j * 2
            pltpu.async_copy(
                gs_in.at[src_sf.at[pl.ds((k0 + 1) * SUBC, SUBC)]], rows1, gs1)
            drain(rows, gs0)
            pltpu.sync_copy(rows, acc_sh.at[dst_sf.at[pl.ds(k0 * SUBC, SUBC)]],
                            add=True)

            @pl.when(k0 + 2 < NK)
            def _():
                pltpu.async_copy(
                    gs_in.at[src_sf.at[pl.ds((k0 + 2) * SUBC, SUBC)]],
                    rows, gs0)
            drain(rows1, gs1)
            pltpu.sync_copy(rows1,
                            acc_sh.at[dst_sf.at[pl.ds((k0 + 1) * SUBC, SUBC)]],
                            add=True)
            return ()
        lax.fori_loop(0, (NK - 1) // 2, body2, ())
        drain(rows, gs0)
        pltpu.sync_copy(rows, acc_sh.at[dst_sf.at[pl.ds((NK - 1) * SUBC, SUBC)]],
                        add=True)
        plsc.subcore_barrier()

        pltpu.sync_copy(acc_sh.at[pl.ds(sid * TECN, TECN), :],
                        e_out.at[pl.ds(cid * NP + sid * TECN, TECN), :])

    return kern


def _xw_body(x_ref, w_ref, d1_ref, o_ref):
    o_ref[...] = d1_ref[...] * jnp.dot(x_ref[...], w_ref[...],
                                       preferred_element_type=jnp.float32)


def _mk_xw():
    return pl.pallas_call(
        _xw_body,
        grid=(NP // 1024,),
        in_specs=[
            pl.BlockSpec((1024, F), lambda i: (i, 0)),
            pl.BlockSpec((F, F), lambda i: (0, 0)),
            pl.BlockSpec((1024, 1), lambda i: (i, 0)),
        ],
        out_specs=pl.BlockSpec((1024, F), lambda i: (i, 0)),
        out_shape=jax.ShapeDtypeStruct((NP, F), jnp.float32),
    )


def _final_body(gs_ref, e_ref, d1_ref, bt_ref, p_ref, b1_ref, w2_ref, b2_ref,
                o_ref, accp, accs, accc):
    i = pl.program_id(0)

    @pl.when(i == 0)
    def _():
        accp[...] = jnp.zeros_like(accp)
        accs[...] = jnp.zeros_like(accs)
        accc[...] = jnp.zeros_like(accc)

    gsb = gs_ref[...]
    eb = e_ref[0] + e_ref[1]
    d1 = d1_ref[...]
    hb = jax.nn.relu(d1 * (gsb + eb) + b1_ref[...])
    pb = p_ref[0] + p_ref[1]
    msel = (lax.broadcasted_iota(jnp.int32, (1024, NG), 1)
            == bt_ref[...]).astype(jnp.float32)
    dn = (((0,), (0,)), ((), ()))
    dh = d1 * hb
    accp[...] += jnp.dot(pb, dh, preferred_element_type=jnp.float32)
    accs[...] += lax.dot_general(msel, d1 * dh, dn,
                                 preferred_element_type=jnp.float32)
    accc[...] += lax.dot_general(msel, jnp.ones((1024, F), jnp.float32), dn,
                                 preferred_element_type=jnp.float32)

    @pl.when(i == NP // 1024 - 1)
    def _():
        pooled = (accp[...] + accs[...]) / jnp.maximum(accc[...], 1.0)
        logits = jnp.dot(pooled, w2_ref[...],
                         preferred_element_type=jnp.float32) + b2_ref[...]
        m = jnp.max(logits, axis=1, keepdims=True)
        s = logits - m
        o_ref[...] = s - jnp.log(jnp.sum(jnp.exp(s), axis=1, keepdims=True))


def _mk_final():
    nb = NP // 1024
    return pl.pallas_call(
        _final_body,
        grid=(nb,),
        in_specs=[
            pl.BlockSpec((1024, F), lambda i: (i, 0)),
            pl.BlockSpec((2, 1024, F), lambda i: (0, i, 0)),
            pl.BlockSpec((1024, 1), lambda i: (i, 0)),
            pl.BlockSpec((1024, 1), lambda i: (i, 0)),
            pl.BlockSpec((2, NG, 1024), lambda i: (0, 0, i)),
            pl.BlockSpec((1, F), lambda i: (0, 0)),
            pl.BlockSpec((F, F), lambda i: (0, 0)),
            pl.BlockSpec((1, F), lambda i: (0, 0)),
        ],
        out_specs=pl.BlockSpec((NG, F), lambda i: (0, 0)),
        out_shape=jax.ShapeDtypeStruct((NG, F), jnp.float32),
        scratch_shapes=[
            pltpu.VMEM((NG, F), jnp.float32),
            pltpu.VMEM((NG, F), jnp.float32),
            pltpu.VMEM((NG, F), jnp.float32),
        ],
    )


_edge_stats = _mk_edge_stats()
_aggregate = _mk_aggregate()
_xw = _mk_xw()
_final = _mk_final()


def kernel(x, edge_index, batch, W1, b1, W2, b2):
    eif = edge_index.reshape(2 * E)
    x_pad = jnp.pad(x, ((0, NP - N), (0, 0)))
    batch_pad = jnp.pad(batch, (0, NP - N), constant_values=NG)

    dinv, p_part = _edge_stats(eif, batch_pad)
    d1col = dinv[:NP][:, None]
    gs = _xw(x_pad, W1, d1col)
    e_part = _aggregate(eif, gs)

    return _final(gs, e_part.reshape(2, NP, F), d1col, batch_pad[:, None],
                  p_part.reshape(2, NG, NP),
                  b1.reshape(1, F), W2, b2.reshape(1, F))
